# baseline (speedup 1.0000x reference)
import functools

import jax
import jax.numpy as jnp
from jax import lax
from jax.experimental import pallas as pl
from jax.experimental.pallas import tpu as pltpu
from jax.experimental.pallas import tpu_sc as plsc

GAMMA = 12.0
HIDDEN = 128
BATCH = 4096

_info = plsc.get_sparse_core_info()
_NC, _NS = _info.num_cores, _info.num_subcores
_NW = _NC * _NS
_BPW = BATCH // _NW
_NSTAGE = 4
_RPS = _BPW // _NSTAGE


def _make_kernel():
    mesh = plsc.VectorSubcoreMesh(core_axis_name="c", subcore_axis_name="s")

    @functools.partial(
        pl.kernel,
        mesh=mesh,
        out_type=jax.ShapeDtypeStruct((BATCH,), jnp.float32),
        scratch_types=(
            [pltpu.VMEM((_BPW,), jnp.int32) for _ in range(3)]
            + [pltpu.VMEM((_BPW,), jnp.int32) for _ in range(3)]
            + [pltpu.VMEM((_RPS, HIDDEN), jnp.float32) for _ in range(_NSTAGE)]
            + [pltpu.VMEM((_RPS, HIDDEN), jnp.float32) for _ in range(_NSTAGE)]
            + [pltpu.VMEM((_BPW,), jnp.float32),
               pltpu.VMEM((16 * 48,), jnp.float32)]
            + [pltpu.SemaphoreType.DMA for _ in range(3 + 3 * _NSTAGE)]
        ),
    )
    def transe(ent_hbm, rel_hbm, sample_hbm, out_hbm,
               hpos, rpos, tpos, hidx, ridx, tidx,
               hr0, hr1, hr2, hr3, t0, t1, t2, t3,
               scores, w,
               sem_ih, sem_ir, sem_it, *sems):
        hrbufs = (hr0, hr1, hr2, hr3)
        tbufs = (t0, t1, t2, t3)
        sem_h = sems[0:_NSTAGE]
        sem_r = sems[_NSTAGE:2 * _NSTAGE]
        sem_t = sems[2 * _NSTAGE:3 * _NSTAGE]

        wid = lax.axis_index("s") * _NC + lax.axis_index("c")
        base = wid * _BPW
        lane = lax.iota(jnp.int32, 16)

        for c in range(_BPW // 16):
            v = (lane + (base + 16 * c)) * 3
            hpos[pl.ds(16 * c, 16)] = v
            rpos[pl.ds(16 * c, 16)] = v + 1
            tpos[pl.ds(16 * c, 16)] = v + 2
        cih = pltpu.async_copy(sample_hbm.at[hpos], hidx, sem_ih)
        cir = pltpu.async_copy(sample_hbm.at[rpos], ridx, sem_ir)
        cit = pltpu.async_copy(sample_hbm.at[tpos], tidx, sem_it)
        cih.wait()
        cir.wait()
        cit.wait()

        cps_h = [pltpu.async_copy(ent_hbm.at[hidx.at[pl.ds(s * _RPS, _RPS)]],
                                  hrbufs[s], sem_h[s]) for s in range(_NSTAGE)]
        cps_t = [pltpu.async_copy(ent_hbm.at[tidx.at[pl.ds(s * _RPS, _RPS)]],
                                  tbufs[s], sem_t[s]) for s in range(_NSTAGE)]
        cps_h[0].wait()
        cps_r = [pltpu.async_copy(rel_hbm.at[ridx.at[pl.ds(0, _RPS)]],
                                  hrbufs[0], sem_r[0], add=True)]

        for s in range(_NSTAGE):
            cps_r[s].wait()
            cps_t[s].wait()
            if s + 1 < _NSTAGE:
                cps_h[s + 1].wait()
                cps_r.append(pltpu.async_copy(
                    rel_hbm.at[ridx.at[pl.ds((s + 1) * _RPS, _RPS)]],
                    hrbufs[s + 1], sem_r[s + 1], add=True))
            hrbuf, tbuf = hrbufs[s], tbufs[s]

            def gbody(g, _, hrbuf=hrbuf, tbuf=tbuf, s=s):
                res = jnp.zeros((16,), jnp.float32)
                for j in range(16):
                    b = g * 16 + j
                    acc = jnp.zeros((16,), jnp.float32)
                    for c in range(HIDDEN // 16):
                        hv = hrbuf[b, pl.ds(c * 16, 16)]
                        tv = tbuf[b, pl.ds(c * 16, 16)]
                        acc = acc + jnp.abs(hv - tv)
                    p = j * 48 + 16
                    w[pl.ds(p, 16)] = acc
                    r1 = acc + w[pl.ds(p + 8, 16)]
                    w[pl.ds(p, 16)] = r1
                    r2 = r1 + w[pl.ds(p + 4, 16)]
                    w[pl.ds(p, 16)] = r2
                    r3 = r2 + w[pl.ds(p + 2, 16)]
                    w[pl.ds(p, 16)] = r3
                    r4 = r3 + w[pl.ds(p + 1, 16)]
                    w[pl.ds(p, 16)] = r4
                    f = w[pl.ds(p - j, 16)]
                    res = jnp.where(lane == j, f, res)
                scores[pl.ds(s * _RPS + g * 16, 16)] = GAMMA - res
                return 0

            lax.fori_loop(0, _RPS // 16, gbody, 0)

        pltpu.sync_copy(scores, out_hbm.at[pl.ds(base, _BPW)])

    return transe


_transe = _make_kernel()


def kernel(sample, entity_embedding, relation_embedding):
    scores = _transe(entity_embedding, relation_embedding,
                     jnp.reshape(sample, (-1,)))
    return scores[:, None]
